# baseline (speedup 1.0000x reference)
import functools

import jax
import jax.numpy as jnp
from jax import lax
from jax.experimental import pallas as pl
from jax.experimental.pallas import tpu as pltpu
from jax.experimental.pallas import tpu_sc as plsc

N = 10000
E = 320000
D = 128
NPAD = 10240
NC = 2
NS = 16
NW = NC * NS
CHUNK = 128
CPW = 80
EPAD = NW * CPW * CHUNK
ROWS_PER_TILE = NPAD // NS

_mesh = plsc.VectorSubcoreMesh(
    core_axis_name="c", subcore_axis_name="s", num_cores=NC, num_subcores=NS)


def _unpack_chunk(idxp, j, slot_s, slot_d):
    row = idxp.at[j]
    for k in range(CHUNK // 16):
        pk = row[pl.ds(k * 16, 16)]
        slot_s[pl.ds(k * 16, 16)] = jnp.bitwise_and(pk, 0xFFFF)
        slot_d[pl.ds(k * 16, 16)] = lax.shift_right_logical(pk, 16)


@functools.partial(
    pl.kernel,
    out_type=jax.ShapeDtypeStruct((2, NC, NPAD, 16), jnp.float32),
    mesh=_mesh,
    scratch_types=[
        pltpu.VMEM((CPW, CHUNK), jnp.int32),
        pltpu.VMEM((CHUNK,), jnp.int32),
        pltpu.VMEM((CHUNK,), jnp.int32),
        pltpu.VMEM((CHUNK, 16), jnp.float32),
        pltpu.VMEM_SHARED((NPAD, 16), jnp.float32),
        pltpu.VMEM_SHARED((NPAD, 16), jnp.float32),
    ],
)
def _sc_degrees(idxp_hbm, zeros16, out, idxp, slot_s, slot_d, ones_v,
                acc_o, acc_i):
    cid = lax.axis_index("c")
    sid = lax.axis_index("s")
    w = sid * NC + cid

    one = jnp.full((16,), 1.0, jnp.float32)
    for r in range(CHUNK):
        ones_v[r, :] = one
    base = sid * ROWS_PER_TILE
    pltpu.sync_copy(zeros16, acc_o.at[pl.ds(base, ROWS_PER_TILE)])
    pltpu.sync_copy(zeros16, acc_i.at[pl.ds(base, ROWS_PER_TILE)])
    pltpu.sync_copy(idxp_hbm.at[pl.ds(w * CPW, CPW)], idxp)
    plsc.subcore_barrier()

    def body(j, _):
        _unpack_chunk(idxp, j, slot_s, slot_d)
        pltpu.sync_copy(ones_v, acc_o.at[slot_s], add=True)
        pltpu.sync_copy(ones_v, acc_i.at[slot_d], add=True)
        return _

    lax.fori_loop(0, CPW, body, 0)
    plsc.subcore_barrier()

    pltpu.sync_copy(acc_o.at[pl.ds(base, ROWS_PER_TILE)],
                    out.at[0, cid, pl.ds(base, ROWS_PER_TILE)])
    pltpu.sync_copy(acc_i.at[pl.ds(base, ROWS_PER_TILE)],
                    out.at[1, cid, pl.ds(base, ROWS_PER_TILE)])


@functools.partial(
    pl.kernel,
    out_type=jax.ShapeDtypeStruct((NC, NPAD, D), jnp.float32),
    mesh=_mesh,
    scratch_types=[
        pltpu.VMEM((CPW, CHUNK), jnp.int32),
        [pltpu.VMEM((CHUNK,), jnp.int32)] * 4,
        [pltpu.VMEM((CHUNK,), jnp.int32)] * 4,
        [pltpu.VMEM((CHUNK, D), jnp.float32)] * 2,
        pltpu.VMEM_SHARED((NPAD, D), jnp.float32),
        [pltpu.SemaphoreType.DMA] * 2,
    ],
)
def _sc_gather_scatter(hs, idxp_hbm, zeros, out, idxp, ss, sd, rows, acc,
                       sems):
    cid = lax.axis_index("c")
    sid = lax.axis_index("s")
    w = sid * NC + cid
    base = sid * ROWS_PER_TILE

    pltpu.sync_copy(zeros, acc.at[pl.ds(base, ROWS_PER_TILE)])
    pltpu.sync_copy(idxp_hbm.at[pl.ds(w * CPW, CPW)], idxp)
    plsc.subcore_barrier()

    def body(j, _):
        _unpack_chunk(idxp, j, ss[0], sd[0])
        pltpu.async_copy(hs.at[ss[0]], rows[0], sems[0]).wait()
        pltpu.sync_copy(rows[0], acc.at[sd[0]], add=True)
        return _

    lax.fori_loop(0, CPW, body, 0)
    plsc.subcore_barrier()

    base = sid * ROWS_PER_TILE
    pltpu.sync_copy(acc.at[pl.ds(base, ROWS_PER_TILE)],
                    out.at[cid, pl.ds(base, ROWS_PER_TILE)])


_BLK = 1024
_NBLK = NPAD // _BLK


def _tc_embed_body(dp_ref, h_ref, w_ref, b_ref, h0_ref, hs_ref, a_ref, b8_ref):
    deg_o = dp_ref[0, 0, :, 0:1] + dp_ref[0, 1, :, 0:1]
    deg_i = dp_ref[1, 0, :, 0:1] + dp_ref[1, 1, :, 0:1]
    a = lax.rsqrt(jnp.maximum(deg_o, 1.0))
    b = lax.rsqrt(jnp.maximum(deg_i, 1.0))
    h0 = jnp.dot(h_ref[...], w_ref[...], preferred_element_type=jnp.float32)
    h0 = h0 + b_ref[...]
    h0_ref[...] = h0
    hs_ref[...] = a * h0
    a_ref[...] = jnp.broadcast_to(a, (_BLK, 8))
    b8_ref[...] = jnp.broadcast_to(b, (_BLK, 8))


def _tc_embed(deg_parts, h_pad, W, bvec):
    return pl.pallas_call(
        _tc_embed_body,
        grid=(_NBLK,),
        in_specs=[
            pl.BlockSpec((2, NC, _BLK, 16), lambda i: (0, 0, i, 0)),
            pl.BlockSpec((_BLK, D), lambda i: (i, 0)),
            pl.BlockSpec((D, D), lambda i: (0, 0)),
            pl.BlockSpec((1, D), lambda i: (0, 0)),
        ],
        out_specs=[
            pl.BlockSpec((_BLK, D), lambda i: (i, 0)),
            pl.BlockSpec((_BLK, D), lambda i: (i, 0)),
            pl.BlockSpec((_BLK, 8), lambda i: (i, 0)),
            pl.BlockSpec((_BLK, 8), lambda i: (i, 0)),
        ],
        out_shape=[
            jax.ShapeDtypeStruct((NPAD, D), jnp.float32),
            jax.ShapeDtypeStruct((NPAD, D), jnp.float32),
            jax.ShapeDtypeStruct((NPAD, 8), jnp.float32),
            jax.ShapeDtypeStruct((NPAD, 8), jnp.float32),
        ],
    )(deg_parts, h_pad, W, bvec)


def _tc_layer_a_body(p_ref, b8_ref, w_ref, bias_ref, y_ref, s1_ref, s2_ref):
    i = pl.program_id(0)
    t = (p_ref[0, :, :] + p_ref[1, :, :]) * b8_ref[:, 0:1]
    y = jnp.dot(t, w_ref[...], preferred_element_type=jnp.float32) + bias_ref[...]
    rows = lax.broadcasted_iota(jnp.int32, (_BLK, 1), 0) + i * _BLK
    y = jnp.where(rows < N, y, 0.0)
    y_ref[...] = y

    @pl.when(i == 0)
    def _():
        s1_ref[...] = jnp.zeros_like(s1_ref)
        s2_ref[...] = jnp.zeros_like(s2_ref)

    s1_ref[...] += jnp.sum(y, axis=0, keepdims=True)
    s2_ref[...] += jnp.sum(y * y, axis=0, keepdims=True)


def _tc_layer_a(parts, b8, W, bias):
    return pl.pallas_call(
        _tc_layer_a_body,
        grid=(_NBLK,),
        in_specs=[
            pl.BlockSpec((NC, _BLK, D), lambda i: (0, i, 0)),
            pl.BlockSpec((_BLK, 8), lambda i: (i, 0)),
            pl.BlockSpec((D, D), lambda i: (0, 0)),
            pl.BlockSpec((1, D), lambda i: (0, 0)),
        ],
        out_specs=[
            pl.BlockSpec((_BLK, D), lambda i: (i, 0)),
            pl.BlockSpec((1, D), lambda i: (0, 0)),
            pl.BlockSpec((1, D), lambda i: (0, 0)),
        ],
        out_shape=[
            jax.ShapeDtypeStruct((NPAD, D), jnp.float32),
            jax.ShapeDtypeStruct((1, D), jnp.float32),
            jax.ShapeDtypeStruct((1, D), jnp.float32),
        ],
    )(parts, b8, W, bias)


def _tc_layer_b_body(y_ref, h_ref, s1_ref, s2_ref, g_ref, bt_ref, a8_ref,
                     hn_ref, hs_ref):
    mu = s1_ref[...] / N
    var = s2_ref[...] / N - mu * mu
    inv = g_ref[...] * lax.rsqrt(var + 1e-5)
    hn = (y_ref[...] - mu) * inv + bt_ref[...]
    hn = jnp.maximum(hn, 0.0)
    h_new = h_ref[...] + hn
    hn_ref[...] = h_new
    hs_ref[...] = a8_ref[:, 0:1] * h_new


def _tc_layer_b(y, h, s1, s2, gamma, beta, a8):
    return pl.pallas_call(
        _tc_layer_b_body,
        grid=(_NBLK,),
        in_specs=[
            pl.BlockSpec((_BLK, D), lambda i: (i, 0)),
            pl.BlockSpec((_BLK, D), lambda i: (i, 0)),
            pl.BlockSpec((1, D), lambda i: (0, 0)),
            pl.BlockSpec((1, D), lambda i: (0, 0)),
            pl.BlockSpec((1, D), lambda i: (0, 0)),
            pl.BlockSpec((1, D), lambda i: (0, 0)),
            pl.BlockSpec((_BLK, 8), lambda i: (i, 0)),
        ],
        out_specs=[
            pl.BlockSpec((_BLK, D), lambda i: (i, 0)),
            pl.BlockSpec((_BLK, D), lambda i: (i, 0)),
        ],
        out_shape=[
            jax.ShapeDtypeStruct((NPAD, D), jnp.float32),
            jax.ShapeDtypeStruct((NPAD, D), jnp.float32),
        ],
    )(y, h, s1, s2, gamma, beta, a8)


def _tc_head_body(h_ref, w0_ref, b0_ref, w1_ref, b1_ref, w2_ref, b2_ref,
                  out_ref, acc_ref):
    i = pl.program_id(0)

    @pl.when(i == 0)
    def _():
        acc_ref[...] = jnp.zeros_like(acc_ref)

    rows = lax.broadcasted_iota(jnp.int32, (_BLK, 1), 0) + i * _BLK
    hm = jnp.where(rows < N, h_ref[...], 0.0)
    acc_ref[...] += jnp.sum(hm, axis=0, keepdims=True)

    @pl.when(i == _NBLK - 1)
    def _():
        hg = acc_ref[...] / N
        y = jnp.dot(hg, w0_ref[...], preferred_element_type=jnp.float32)
        y = jnp.maximum(y + b0_ref[...], 0.0)
        y = jnp.dot(y, w1_ref[...], preferred_element_type=jnp.float32)
        y = jnp.maximum(y + b1_ref[...], 0.0)
        y = jnp.dot(y, w2_ref[...], preferred_element_type=jnp.float32)
        out_ref[...] = y + b2_ref[...]


def _tc_head(h, W0, b0, W1, b1, W2, b2):
    return pl.pallas_call(
        _tc_head_body,
        grid=(_NBLK,),
        in_specs=[
            pl.BlockSpec((_BLK, D), lambda i: (i, 0)),
            pl.BlockSpec(W0.shape, lambda i: (0, 0)),
            pl.BlockSpec((1, W0.shape[1]), lambda i: (0, 0)),
            pl.BlockSpec(W1.shape, lambda i: (0, 0)),
            pl.BlockSpec((1, W1.shape[1]), lambda i: (0, 0)),
            pl.BlockSpec(W2.shape, lambda i: (0, 0)),
            pl.BlockSpec((1, W2.shape[1]), lambda i: (0, 0)),
        ],
        out_specs=pl.BlockSpec((1, W2.shape[1]), lambda i: (0, 0)),
        out_shape=jax.ShapeDtypeStruct((1, W2.shape[1]), jnp.float32),
        scratch_shapes=[pltpu.VMEM((1, D), jnp.float32)],
    )(h, W0, b0, W1, b1, W2, b2)


def kernel(h, e, edge_index, W_emb_h, b_emb_h, W_emb_e, b_emb_e, W_conv,
           b_conv, gamma, beta, W_mlp0, b_mlp0, W_mlp1, b_mlp1, W_mlp2, b_mlp2):
    del e, W_emb_e, b_emb_e

    packed = edge_index[0] | (edge_index[1] << 16)
    pad = jnp.full((EPAD - E,), N | (N << 16), jnp.int32)
    idxp = jnp.concatenate([packed, pad]).reshape(EPAD // CHUNK, CHUNK)
    h_pad = jnp.pad(h, ((0, NPAD - N), (0, 0)))
    zeros = jnp.zeros((ROWS_PER_TILE, D), jnp.float32)
    zeros16 = jnp.zeros((ROWS_PER_TILE, 16), jnp.float32)

    deg_parts = _sc_degrees(idxp, zeros16)
    hcur, hs, a8, b8 = _tc_embed(deg_parts, h_pad, W_emb_h,
                                 b_emb_h.reshape(1, D))
    for l in range(W_conv.shape[0]):
        parts = _sc_gather_scatter(hs, idxp, zeros)
        y, s1, s2 = _tc_layer_a(parts, b8, W_conv[l], b_conv[l].reshape(1, D))
        hcur, hs = _tc_layer_b(y, hcur, s1, s2, gamma[l].reshape(1, D),
                               beta[l].reshape(1, D), a8)
    return _tc_head(hcur, W_mlp0, b_mlp0.reshape(1, -1), W_mlp1,
                    b_mlp1.reshape(1, -1), W_mlp2, b_mlp2.reshape(1, -1))


# speedup vs baseline: 1.1248x; 1.1248x over previous
import functools

import jax
import jax.numpy as jnp
from jax import lax
from jax.experimental import pallas as pl
from jax.experimental.pallas import tpu as pltpu
from jax.experimental.pallas import tpu_sc as plsc

N = 10000
E = 320000
D = 128
NPAD = 10240
NC = 2
NS = 16
NW = NC * NS
CHUNK = 128
CPW = 80
EPAD = NW * CPW * CHUNK
ROWS_PER_TILE = NPAD // NS

_mesh = plsc.VectorSubcoreMesh(
    core_axis_name="c", subcore_axis_name="s", num_cores=NC, num_subcores=NS)


def _unpack_chunk(idxp, j, slot_s, slot_d):
    row = idxp.at[j]
    for k in range(CHUNK // 16):
        pk = row[pl.ds(k * 16, 16)]
        slot_s[pl.ds(k * 16, 16)] = jnp.bitwise_and(pk, 0xFFFF)
        slot_d[pl.ds(k * 16, 16)] = lax.shift_right_logical(pk, 16)


@functools.partial(
    pl.kernel,
    out_type=jax.ShapeDtypeStruct((2, NC, NPAD, 16), jnp.float32),
    mesh=_mesh,
    scratch_types=[
        pltpu.VMEM((CPW, CHUNK), jnp.int32),
        pltpu.VMEM((CHUNK,), jnp.int32),
        pltpu.VMEM((CHUNK,), jnp.int32),
        pltpu.VMEM((CHUNK, 16), jnp.float32),
        pltpu.VMEM_SHARED((NPAD, 16), jnp.float32),
        pltpu.VMEM_SHARED((NPAD, 16), jnp.float32),
    ],
)
def _sc_degrees(idxp_hbm, zeros16, out, idxp, slot_s, slot_d, ones_v,
                acc_o, acc_i):
    cid = lax.axis_index("c")
    sid = lax.axis_index("s")
    w = sid * NC + cid

    one = jnp.full((16,), 1.0, jnp.float32)
    for r in range(CHUNK):
        ones_v[r, :] = one
    base = sid * ROWS_PER_TILE
    pltpu.sync_copy(zeros16, acc_o.at[pl.ds(base, ROWS_PER_TILE)])
    pltpu.sync_copy(zeros16, acc_i.at[pl.ds(base, ROWS_PER_TILE)])
    pltpu.sync_copy(idxp_hbm.at[pl.ds(w * CPW, CPW)], idxp)
    plsc.subcore_barrier()

    def body(j, _):
        _unpack_chunk(idxp, j, slot_s, slot_d)
        pltpu.sync_copy(ones_v, acc_o.at[slot_s], add=True)
        pltpu.sync_copy(ones_v, acc_i.at[slot_d], add=True)
        return _

    lax.fori_loop(0, CPW, body, 0)
    plsc.subcore_barrier()

    pltpu.sync_copy(acc_o.at[pl.ds(base, ROWS_PER_TILE)],
                    out.at[0, cid, pl.ds(base, ROWS_PER_TILE)])
    pltpu.sync_copy(acc_i.at[pl.ds(base, ROWS_PER_TILE)],
                    out.at[1, cid, pl.ds(base, ROWS_PER_TILE)])


@functools.partial(
    pl.kernel,
    out_type=jax.ShapeDtypeStruct((NC, NPAD, D), jnp.float32),
    mesh=_mesh,
    scratch_types=[
        pltpu.VMEM((CPW, CHUNK), jnp.int32),
        [pltpu.VMEM((CHUNK,), jnp.int32)] * 4,
        [pltpu.VMEM((CHUNK,), jnp.int32)] * 4,
        [pltpu.VMEM((CHUNK, D), jnp.float32)] * 2,
        pltpu.VMEM_SHARED((NPAD, D), jnp.float32),
        [pltpu.SemaphoreType.DMA] * 2,
    ],
)
def _sc_gather_scatter(hs, idxp_hbm, zeros, out, idxp, ss, sd, rows, acc,
                       sems):
    cid = lax.axis_index("c")
    sid = lax.axis_index("s")
    w = sid * NC + cid
    base = sid * ROWS_PER_TILE

    pltpu.sync_copy(zeros, acc.at[pl.ds(base, ROWS_PER_TILE)])
    pltpu.sync_copy(idxp_hbm.at[pl.ds(w * CPW, CPW)], idxp)
    plsc.subcore_barrier()

    for j in range(2):
        _unpack_chunk(idxp, j, ss[j], sd[j])
        pltpu.async_copy(hs.at[ss[j]], rows[j], sems[j])

    def step(j, b, issue=True):
        s = b % 4
        r = b % 2
        pltpu.make_async_copy(hs.at[ss[s]], rows[r], sems[r]).wait()
        pltpu.sync_copy(rows[r], acc.at[sd[s]], add=True)
        if issue:
            s2 = (b + 2) % 4
            _unpack_chunk(idxp, j + 2, ss[s2], sd[s2])
            pltpu.async_copy(hs.at[ss[s2]], rows[r], sems[r])

    def group(g, _):
        for b in range(4):
            step(g * 4 + b, b)
        return _

    lax.fori_loop(0, CPW // 4 - 1, group, 0)
    for b in range(4):
        step(CPW - 4 + b, b, issue=(b < 2))
    plsc.subcore_barrier()

    base = sid * ROWS_PER_TILE
    pltpu.sync_copy(acc.at[pl.ds(base, ROWS_PER_TILE)],
                    out.at[cid, pl.ds(base, ROWS_PER_TILE)])


_BLK = 1024
_NBLK = NPAD // _BLK


def _tc_embed_body(dp_ref, h_ref, w_ref, b_ref, h0_ref, hs_ref, a_ref, b8_ref):
    deg_o = dp_ref[0, 0, :, 0:1] + dp_ref[0, 1, :, 0:1]
    deg_i = dp_ref[1, 0, :, 0:1] + dp_ref[1, 1, :, 0:1]
    a = lax.rsqrt(jnp.maximum(deg_o, 1.0))
    b = lax.rsqrt(jnp.maximum(deg_i, 1.0))
    h0 = jnp.dot(h_ref[...], w_ref[...], preferred_element_type=jnp.float32)
    h0 = h0 + b_ref[...]
    h0_ref[...] = h0
    hs_ref[...] = a * h0
    a_ref[...] = jnp.broadcast_to(a, (_BLK, 8))
    b8_ref[...] = jnp.broadcast_to(b, (_BLK, 8))


def _tc_embed(deg_parts, h_pad, W, bvec):
    return pl.pallas_call(
        _tc_embed_body,
        grid=(_NBLK,),
        in_specs=[
            pl.BlockSpec((2, NC, _BLK, 16), lambda i: (0, 0, i, 0)),
            pl.BlockSpec((_BLK, D), lambda i: (i, 0)),
            pl.BlockSpec((D, D), lambda i: (0, 0)),
            pl.BlockSpec((1, D), lambda i: (0, 0)),
        ],
        out_specs=[
            pl.BlockSpec((_BLK, D), lambda i: (i, 0)),
            pl.BlockSpec((_BLK, D), lambda i: (i, 0)),
            pl.BlockSpec((_BLK, 8), lambda i: (i, 0)),
            pl.BlockSpec((_BLK, 8), lambda i: (i, 0)),
        ],
        out_shape=[
            jax.ShapeDtypeStruct((NPAD, D), jnp.float32),
            jax.ShapeDtypeStruct((NPAD, D), jnp.float32),
            jax.ShapeDtypeStruct((NPAD, 8), jnp.float32),
            jax.ShapeDtypeStruct((NPAD, 8), jnp.float32),
        ],
    )(deg_parts, h_pad, W, bvec)


def _tc_layer_a_body(p_ref, b8_ref, w_ref, bias_ref, y_ref, s1_ref, s2_ref):
    i = pl.program_id(0)
    t = (p_ref[0, :, :] + p_ref[1, :, :]) * b8_ref[:, 0:1]
    y = jnp.dot(t, w_ref[...], preferred_element_type=jnp.float32) + bias_ref[...]
    rows = lax.broadcasted_iota(jnp.int32, (_BLK, 1), 0) + i * _BLK
    y = jnp.where(rows < N, y, 0.0)
    y_ref[...] = y

    @pl.when(i == 0)
    def _():
        s1_ref[...] = jnp.zeros_like(s1_ref)
        s2_ref[...] = jnp.zeros_like(s2_ref)

    s1_ref[...] += jnp.sum(y, axis=0, keepdims=True)
    s2_ref[...] += jnp.sum(y * y, axis=0, keepdims=True)


def _tc_layer_a(parts, b8, W, bias):
    return pl.pallas_call(
        _tc_layer_a_body,
        grid=(_NBLK,),
        in_specs=[
            pl.BlockSpec((NC, _BLK, D), lambda i: (0, i, 0)),
            pl.BlockSpec((_BLK, 8), lambda i: (i, 0)),
            pl.BlockSpec((D, D), lambda i: (0, 0)),
            pl.BlockSpec((1, D), lambda i: (0, 0)),
        ],
        out_specs=[
            pl.BlockSpec((_BLK, D), lambda i: (i, 0)),
            pl.BlockSpec((1, D), lambda i: (0, 0)),
            pl.BlockSpec((1, D), lambda i: (0, 0)),
        ],
        out_shape=[
            jax.ShapeDtypeStruct((NPAD, D), jnp.float32),
            jax.ShapeDtypeStruct((1, D), jnp.float32),
            jax.ShapeDtypeStruct((1, D), jnp.float32),
        ],
    )(parts, b8, W, bias)


def _tc_layer_b_body(y_ref, h_ref, s1_ref, s2_ref, g_ref, bt_ref, a8_ref,
                     hn_ref, hs_ref):
    mu = s1_ref[...] / N
    var = s2_ref[...] / N - mu * mu
    inv = g_ref[...] * lax.rsqrt(var + 1e-5)
    hn = (y_ref[...] - mu) * inv + bt_ref[...]
    hn = jnp.maximum(hn, 0.0)
    h_new = h_ref[...] + hn
    hn_ref[...] = h_new
    hs_ref[...] = a8_ref[:, 0:1] * h_new


def _tc_layer_b(y, h, s1, s2, gamma, beta, a8):
    return pl.pallas_call(
        _tc_layer_b_body,
        grid=(_NBLK,),
        in_specs=[
            pl.BlockSpec((_BLK, D), lambda i: (i, 0)),
            pl.BlockSpec((_BLK, D), lambda i: (i, 0)),
            pl.BlockSpec((1, D), lambda i: (0, 0)),
            pl.BlockSpec((1, D), lambda i: (0, 0)),
            pl.BlockSpec((1, D), lambda i: (0, 0)),
            pl.BlockSpec((1, D), lambda i: (0, 0)),
            pl.BlockSpec((_BLK, 8), lambda i: (i, 0)),
        ],
        out_specs=[
            pl.BlockSpec((_BLK, D), lambda i: (i, 0)),
            pl.BlockSpec((_BLK, D), lambda i: (i, 0)),
        ],
        out_shape=[
            jax.ShapeDtypeStruct((NPAD, D), jnp.float32),
            jax.ShapeDtypeStruct((NPAD, D), jnp.float32),
        ],
    )(y, h, s1, s2, gamma, beta, a8)


def _tc_head_body(h_ref, w0_ref, b0_ref, w1_ref, b1_ref, w2_ref, b2_ref,
                  out_ref, acc_ref):
    i = pl.program_id(0)

    @pl.when(i == 0)
    def _():
        acc_ref[...] = jnp.zeros_like(acc_ref)

    rows = lax.broadcasted_iota(jnp.int32, (_BLK, 1), 0) + i * _BLK
    hm = jnp.where(rows < N, h_ref[...], 0.0)
    acc_ref[...] += jnp.sum(hm, axis=0, keepdims=True)

    @pl.when(i == _NBLK - 1)
    def _():
        hg = acc_ref[...] / N
        y = jnp.dot(hg, w0_ref[...], preferred_element_type=jnp.float32)
        y = jnp.maximum(y + b0_ref[...], 0.0)
        y = jnp.dot(y, w1_ref[...], preferred_element_type=jnp.float32)
        y = jnp.maximum(y + b1_ref[...], 0.0)
        y = jnp.dot(y, w2_ref[...], preferred_element_type=jnp.float32)
        out_ref[...] = y + b2_ref[...]


def _tc_head(h, W0, b0, W1, b1, W2, b2):
    return pl.pallas_call(
        _tc_head_body,
        grid=(_NBLK,),
        in_specs=[
            pl.BlockSpec((_BLK, D), lambda i: (i, 0)),
            pl.BlockSpec(W0.shape, lambda i: (0, 0)),
            pl.BlockSpec((1, W0.shape[1]), lambda i: (0, 0)),
            pl.BlockSpec(W1.shape, lambda i: (0, 0)),
            pl.BlockSpec((1, W1.shape[1]), lambda i: (0, 0)),
            pl.BlockSpec(W2.shape, lambda i: (0, 0)),
            pl.BlockSpec((1, W2.shape[1]), lambda i: (0, 0)),
        ],
        out_specs=pl.BlockSpec((1, W2.shape[1]), lambda i: (0, 0)),
        out_shape=jax.ShapeDtypeStruct((1, W2.shape[1]), jnp.float32),
        scratch_shapes=[pltpu.VMEM((1, D), jnp.float32)],
    )(h, W0, b0, W1, b1, W2, b2)


def kernel(h, e, edge_index, W_emb_h, b_emb_h, W_emb_e, b_emb_e, W_conv,
           b_conv, gamma, beta, W_mlp0, b_mlp0, W_mlp1, b_mlp1, W_mlp2, b_mlp2):
    del e, W_emb_e, b_emb_e

    packed = edge_index[0] | (edge_index[1] << 16)
    pad = jnp.full((EPAD - E,), N | (N << 16), jnp.int32)
    idxp = jnp.concatenate([packed, pad]).reshape(EPAD // CHUNK, CHUNK)
    h_pad = jnp.pad(h, ((0, NPAD - N), (0, 0)))
    zeros = jnp.zeros((ROWS_PER_TILE, D), jnp.float32)
    zeros16 = jnp.zeros((ROWS_PER_TILE, 16), jnp.float32)

    deg_parts = _sc_degrees(idxp, zeros16)
    hcur, hs, a8, b8 = _tc_embed(deg_parts, h_pad, W_emb_h,
                                 b_emb_h.reshape(1, D))
    for l in range(W_conv.shape[0]):
        parts = _sc_gather_scatter(hs, idxp, zeros)
        y, s1, s2 = _tc_layer_a(parts, b8, W_conv[l], b_conv[l].reshape(1, D))
        hcur, hs = _tc_layer_b(y, hcur, s1, s2, gamma[l].reshape(1, D),
                               beta[l].reshape(1, D), a8)
    return _tc_head(hcur, W_mlp0, b_mlp0.reshape(1, -1), W_mlp1,
                    b_mlp1.reshape(1, -1), W_mlp2, b_mlp2.reshape(1, -1))


# speedup vs baseline: 3.6140x; 3.2129x over previous
import functools

import jax
import jax.numpy as jnp
from jax import lax
from jax.experimental import pallas as pl
from jax.experimental.pallas import tpu as pltpu
from jax.experimental.pallas import tpu_sc as plsc

N = 10000
E = 320000
D = 128
NPAD = 10240
NC = 2
NS = 16
NW = NC * NS
CHUNK = 128
CPW = 80
EPAD = NW * CPW * CHUNK
ROWS_PER_TILE = NPAD // NS

_mesh = plsc.VectorSubcoreMesh(
    core_axis_name="c", subcore_axis_name="s", num_cores=NC, num_subcores=NS)


def _unpack_chunk(idxp, j, slot_s, slot_d):
    row = idxp.at[j]
    for k in range(CHUNK // 16):
        pk = row[pl.ds(k * 16, 16)]
        slot_s[pl.ds(k * 16, 16)] = jnp.bitwise_and(pk, 0xFFFF)
        slot_d[pl.ds(k * 16, 16)] = lax.shift_right_logical(pk, 16)


@functools.partial(
    pl.kernel,
    out_type=jax.ShapeDtypeStruct((2, NC, NPAD, 16), jnp.float32),
    mesh=_mesh,
    scratch_types=[
        pltpu.VMEM((CPW, CHUNK), jnp.int32),
        pltpu.VMEM((CHUNK,), jnp.int32),
        pltpu.VMEM((CHUNK,), jnp.int32),
        pltpu.VMEM((CHUNK, 16), jnp.float32),
        pltpu.VMEM_SHARED((NPAD, 16), jnp.float32),
        pltpu.VMEM_SHARED((NPAD, 16), jnp.float32),
    ],
)
def _sc_degrees(idxp_hbm, zeros16, out, idxp, slot_s, slot_d, ones_v,
                acc_o, acc_i):
    cid = lax.axis_index("c")
    sid = lax.axis_index("s")
    w = sid * NC + cid

    one = jnp.full((16,), 1.0, jnp.float32)
    for r in range(CHUNK):
        ones_v[r, :] = one
    base = sid * ROWS_PER_TILE
    pltpu.sync_copy(zeros16, acc_o.at[pl.ds(base, ROWS_PER_TILE)])
    pltpu.sync_copy(zeros16, acc_i.at[pl.ds(base, ROWS_PER_TILE)])
    pltpu.sync_copy(idxp_hbm.at[pl.ds(w * CPW, CPW)], idxp)
    plsc.subcore_barrier()

    def body(j, _):
        _unpack_chunk(idxp, j, slot_s, slot_d)
        pltpu.sync_copy(ones_v, acc_o.at[slot_s], add=True)
        pltpu.sync_copy(ones_v, acc_i.at[slot_d], add=True)
        return _

    lax.fori_loop(0, CPW, body, 0)
    plsc.subcore_barrier()

    pltpu.sync_copy(acc_o.at[pl.ds(base, ROWS_PER_TILE)],
                    out.at[0, cid, pl.ds(base, ROWS_PER_TILE)])
    pltpu.sync_copy(acc_i.at[pl.ds(base, ROWS_PER_TILE)],
                    out.at[1, cid, pl.ds(base, ROWS_PER_TILE)])


@functools.partial(
    pl.kernel,
    out_type=jax.ShapeDtypeStruct((NC, NPAD, D), jnp.float32),
    mesh=_mesh,
    scratch_types=[
        pltpu.VMEM((CPW, CHUNK), jnp.int32),
        [pltpu.VMEM((CHUNK,), jnp.int32)] * 4,
        [pltpu.VMEM((CHUNK,), jnp.int32)] * 4,
        [pltpu.VMEM((CHUNK, D), jnp.float32)] * 2,
        pltpu.VMEM_SHARED((NPAD, D), jnp.float32),
        [pltpu.SemaphoreType.DMA] * 2,
    ],
)
def _sc_gather_scatter(hs, idxp_hbm, zeros, out, idxp, ss, sd, rows, acc,
                       sems):
    cid = lax.axis_index("c")
    sid = lax.axis_index("s")
    w = sid * NC + cid
    base = sid * ROWS_PER_TILE

    pltpu.sync_copy(zeros, acc.at[pl.ds(base, ROWS_PER_TILE)])
    pltpu.sync_copy(idxp_hbm.at[pl.ds(w * CPW, CPW)], idxp)
    plsc.subcore_barrier()

    for j in range(2):
        _unpack_chunk(idxp, j, ss[j], sd[j])
        pltpu.async_copy(hs.at[ss[j]], rows[j], sems[j])

    def step(j, b, issue=True):
        s = b % 4
        r = b % 2
        pltpu.make_async_copy(hs.at[ss[s]], rows[r], sems[r]).wait()
        pltpu.sync_copy(rows[r], acc.at[sd[s]], add=True)
        if issue:
            s2 = (b + 2) % 4
            _unpack_chunk(idxp, j + 2, ss[s2], sd[s2])
            pltpu.async_copy(hs.at[ss[s2]], rows[r], sems[r])

    def group(g, _):
        for b in range(4):
            step(g * 4 + b, b)
        return _

    lax.fori_loop(0, CPW // 4 - 1, group, 0)
    for b in range(4):
        step(CPW - 4 + b, b, issue=(b < 2))
    plsc.subcore_barrier()

    base = sid * ROWS_PER_TILE
    pltpu.sync_copy(acc.at[pl.ds(base, ROWS_PER_TILE)],
                    out.at[cid, pl.ds(base, ROWS_PER_TILE)])


_BLK = 1024
_NBLK = NPAD // _BLK


def _tc_embed_body(dp_ref, h_ref, w_ref, b_ref, h0_ref, hs_ref, a_ref, b8_ref):
    deg_o = dp_ref[0, 0, :, 0:1] + dp_ref[0, 1, :, 0:1]
    deg_i = dp_ref[1, 0, :, 0:1] + dp_ref[1, 1, :, 0:1]
    a = lax.rsqrt(jnp.maximum(deg_o, 1.0))
    b = lax.rsqrt(jnp.maximum(deg_i, 1.0))
    h0 = jnp.dot(h_ref[...], w_ref[...], preferred_element_type=jnp.float32)
    h0 = h0 + b_ref[...]
    h0_ref[...] = h0
    hs_ref[...] = a * h0
    a_ref[...] = jnp.broadcast_to(a, (_BLK, 8))
    b8_ref[...] = jnp.broadcast_to(b, (_BLK, 8))


def _tc_embed(deg_parts, h_pad, W, bvec):
    return pl.pallas_call(
        _tc_embed_body,
        grid=(_NBLK,),
        in_specs=[
            pl.BlockSpec((2, NC, _BLK, 16), lambda i: (0, 0, i, 0)),
            pl.BlockSpec((_BLK, D), lambda i: (i, 0)),
            pl.BlockSpec((D, D), lambda i: (0, 0)),
            pl.BlockSpec((1, D), lambda i: (0, 0)),
        ],
        out_specs=[
            pl.BlockSpec((_BLK, D), lambda i: (i, 0)),
            pl.BlockSpec((_BLK, D), lambda i: (i, 0)),
            pl.BlockSpec((_BLK, 8), lambda i: (i, 0)),
            pl.BlockSpec((_BLK, 8), lambda i: (i, 0)),
        ],
        out_shape=[
            jax.ShapeDtypeStruct((NPAD, D), jnp.float32),
            jax.ShapeDtypeStruct((NPAD, D), jnp.float32),
            jax.ShapeDtypeStruct((NPAD, 8), jnp.float32),
            jax.ShapeDtypeStruct((NPAD, 8), jnp.float32),
        ],
    )(deg_parts, h_pad, W, bvec)


def _tc_layer_a_body(p_ref, b8_ref, w_ref, bias_ref, y_ref, s1_ref, s2_ref):
    i = pl.program_id(0)
    t = (p_ref[0, :, :] + p_ref[1, :, :]) * b8_ref[:, 0:1]
    y = jnp.dot(t, w_ref[...], preferred_element_type=jnp.float32) + bias_ref[...]
    rows = lax.broadcasted_iota(jnp.int32, (_BLK, 1), 0) + i * _BLK
    y = jnp.where(rows < N, y, 0.0)
    y_ref[...] = y

    @pl.when(i == 0)
    def _():
        s1_ref[...] = jnp.zeros_like(s1_ref)
        s2_ref[...] = jnp.zeros_like(s2_ref)

    s1_ref[...] += jnp.sum(y, axis=0, keepdims=True)
    s2_ref[...] += jnp.sum(y * y, axis=0, keepdims=True)


def _tc_layer_a(parts, b8, W, bias):
    return pl.pallas_call(
        _tc_layer_a_body,
        grid=(_NBLK,),
        in_specs=[
            pl.BlockSpec((NC, _BLK, D), lambda i: (0, i, 0)),
            pl.BlockSpec((_BLK, 8), lambda i: (i, 0)),
            pl.BlockSpec((D, D), lambda i: (0, 0)),
            pl.BlockSpec((1, D), lambda i: (0, 0)),
        ],
        out_specs=[
            pl.BlockSpec((_BLK, D), lambda i: (i, 0)),
            pl.BlockSpec((1, D), lambda i: (0, 0)),
            pl.BlockSpec((1, D), lambda i: (0, 0)),
        ],
        out_shape=[
            jax.ShapeDtypeStruct((NPAD, D), jnp.float32),
            jax.ShapeDtypeStruct((1, D), jnp.float32),
            jax.ShapeDtypeStruct((1, D), jnp.float32),
        ],
    )(parts, b8, W, bias)


def _tc_layer_b_body(y_ref, h_ref, s1_ref, s2_ref, g_ref, bt_ref, a8_ref,
                     hn_ref, hs_ref):
    mu = s1_ref[...] / N
    var = s2_ref[...] / N - mu * mu
    inv = g_ref[...] * lax.rsqrt(var + 1e-5)
    hn = (y_ref[...] - mu) * inv + bt_ref[...]
    hn = jnp.maximum(hn, 0.0)
    h_new = h_ref[...] + hn
    hn_ref[...] = h_new
    hs_ref[...] = a8_ref[:, 0:1] * h_new


def _tc_layer_b(y, h, s1, s2, gamma, beta, a8):
    return pl.pallas_call(
        _tc_layer_b_body,
        grid=(_NBLK,),
        in_specs=[
            pl.BlockSpec((_BLK, D), lambda i: (i, 0)),
            pl.BlockSpec((_BLK, D), lambda i: (i, 0)),
            pl.BlockSpec((1, D), lambda i: (0, 0)),
            pl.BlockSpec((1, D), lambda i: (0, 0)),
            pl.BlockSpec((1, D), lambda i: (0, 0)),
            pl.BlockSpec((1, D), lambda i: (0, 0)),
            pl.BlockSpec((_BLK, 8), lambda i: (i, 0)),
        ],
        out_specs=[
            pl.BlockSpec((_BLK, D), lambda i: (i, 0)),
            pl.BlockSpec((_BLK, D), lambda i: (i, 0)),
        ],
        out_shape=[
            jax.ShapeDtypeStruct((NPAD, D), jnp.float32),
            jax.ShapeDtypeStruct((NPAD, D), jnp.float32),
        ],
    )(y, h, s1, s2, gamma, beta, a8)


def _tc_head_body(h_ref, w0_ref, b0_ref, w1_ref, b1_ref, w2_ref, b2_ref,
                  out_ref, acc_ref):
    i = pl.program_id(0)

    @pl.when(i == 0)
    def _():
        acc_ref[...] = jnp.zeros_like(acc_ref)

    rows = lax.broadcasted_iota(jnp.int32, (_BLK, 1), 0) + i * _BLK
    hm = jnp.where(rows < N, h_ref[...], 0.0)
    acc_ref[...] += jnp.sum(hm, axis=0, keepdims=True)

    @pl.when(i == _NBLK - 1)
    def _():
        hg = acc_ref[...] / N
        y = jnp.dot(hg, w0_ref[...], preferred_element_type=jnp.float32)
        y = jnp.maximum(y + b0_ref[...], 0.0)
        y = jnp.dot(y, w1_ref[...], preferred_element_type=jnp.float32)
        y = jnp.maximum(y + b1_ref[...], 0.0)
        y = jnp.dot(y, w2_ref[...], preferred_element_type=jnp.float32)
        out_ref[...] = y + b2_ref[...]


def _tc_head(h, W0, b0, W1, b1, W2, b2):
    return pl.pallas_call(
        _tc_head_body,
        grid=(_NBLK,),
        in_specs=[
            pl.BlockSpec((_BLK, D), lambda i: (i, 0)),
            pl.BlockSpec(W0.shape, lambda i: (0, 0)),
            pl.BlockSpec((1, W0.shape[1]), lambda i: (0, 0)),
            pl.BlockSpec(W1.shape, lambda i: (0, 0)),
            pl.BlockSpec((1, W1.shape[1]), lambda i: (0, 0)),
            pl.BlockSpec(W2.shape, lambda i: (0, 0)),
            pl.BlockSpec((1, W2.shape[1]), lambda i: (0, 0)),
        ],
        out_specs=pl.BlockSpec((1, W2.shape[1]), lambda i: (0, 0)),
        out_shape=jax.ShapeDtypeStruct((1, W2.shape[1]), jnp.float32),
        scratch_shapes=[pltpu.VMEM((1, D), jnp.float32)],
    )(h, W0, b0, W1, b1, W2, b2)


def kernel(h, e, edge_index, W_emb_h, b_emb_h, W_emb_e, b_emb_e, W_conv,
           b_conv, gamma, beta, W_mlp0, b_mlp0, W_mlp1, b_mlp1, W_mlp2, b_mlp2):
    del e, W_emb_e, b_emb_e

    packed = edge_index[0] | (edge_index[1] << 16)
    padidx = N + jnp.arange(EPAD - E, dtype=jnp.int32) % (NPAD - N)
    pad = padidx | (padidx << 16)
    idxp = jnp.concatenate([packed, pad]).reshape(EPAD // CHUNK, CHUNK)
    h_pad = jnp.pad(h, ((0, NPAD - N), (0, 0)))
    zeros = jnp.zeros((ROWS_PER_TILE, D), jnp.float32)
    zeros16 = jnp.zeros((ROWS_PER_TILE, 16), jnp.float32)

    deg_parts = _sc_degrees(idxp, zeros16)
    hcur, hs, a8, b8 = _tc_embed(deg_parts, h_pad, W_emb_h,
                                 b_emb_h.reshape(1, D))
    for l in range(W_conv.shape[0]):
        parts = _sc_gather_scatter(hs, idxp, zeros)
        y, s1, s2 = _tc_layer_a(parts, b8, W_conv[l], b_conv[l].reshape(1, D))
        hcur, hs = _tc_layer_b(y, hcur, s1, s2, gamma[l].reshape(1, D),
                               beta[l].reshape(1, D), a8)
    return _tc_head(hcur, W_mlp0, b_mlp0.reshape(1, -1), W_mlp1,
                    b_mlp1.reshape(1, -1), W_mlp2, b_mlp2.reshape(1, -1))
